# Initial kernel scaffold; baseline (speedup 1.0000x reference)
import jax
import jax.numpy as jnp
from jax.experimental import pallas as pl


def kernel(point_cloud, vis_mask, W1, b1, W2, b2, W3, b3):
    raise NotImplementedError("write your pallas kernel here")


# speedup vs baseline: 61.1854x; 61.1854x over previous
import functools

import jax
import jax.numpy as jnp
from jax import lax
from jax.experimental import pallas as pl

K_NN = 20


def _geom_body(ptsr_ref, ptsc_ref, w1_ref, b1_ref, w2_ref, b2_ref,
               w3_ref, b3_ref, out_ref):
    R = ptsr_ref.shape[1]
    N = ptsc_ref.shape[2]
    xr = ptsr_ref[0]
    xc = ptsc_ref[0]

    px = xr[:, 0:1]
    py = xr[:, 1:2]
    pz = xr[:, 2:3]

    dx = xc[0:1, :] - px
    dy = xc[1:2, :] - py
    dz = xc[2:3, :] - pz
    dist = (dx * dx + dy * dy) + dz * dz

    iota = lax.broadcasted_iota(jnp.int32, (R, N), 1)
    big = jnp.float32(jnp.inf)
    sel = jnp.zeros((R, N), dtype=jnp.bool_)
    work = dist
    for r in range(K_NN + 1):
        m = jnp.min(work, axis=1, keepdims=True)
        eq = work == m
        idx = jnp.min(jnp.where(eq, iota, N), axis=1, keepdims=True)
        pick = iota == idx
        work = jnp.where(pick, big, work)
        if r > 0:
            sel = jnp.logical_or(sel, pick)

    zero = jnp.float32(0.0)

    def msum(v):
        return jnp.sum(jnp.where(sel, v, zero), axis=1, keepdims=True)

    kf = jnp.float32(K_NN)
    mlx = msum(dx) / kf
    mly = msum(dy) / kf
    mlz = msum(dz) / kf
    cx = (dx - mlx).astype(jnp.bfloat16).astype(jnp.float32)
    cy = (dy - mly).astype(jnp.bfloat16).astype(jnp.float32)
    cz = (dz - mlz).astype(jnp.bfloat16).astype(jnp.float32)
    a11 = msum(cx * cx)
    a22 = msum(cy * cy)
    a33 = msum(cz * cz)
    a12 = msum(cx * cy)
    a13 = msum(cx * cz)
    a23 = msum(cy * cz)

    scale = jnp.maximum(
        jnp.maximum(jnp.maximum(jnp.abs(a11), jnp.abs(a22)),
                    jnp.maximum(jnp.abs(a33), jnp.abs(a12))),
        jnp.maximum(jnp.abs(a13), jnp.abs(a23)))
    scale = jnp.maximum(scale, jnp.float32(1e-30))
    a11 = a11 / scale
    a22 = a22 / scale
    a33 = a33 / scale
    a12 = a12 / scale
    a13 = a13 / scale
    a23 = a23 / scale

    c2 = a11 + a22 + a33
    c1 = (a11 * a22 + a11 * a33 + a22 * a33) - (a12 * a12 + a13 * a13 + a23 * a23)
    c0 = (a11 * (a22 * a33 - a23 * a23)
          - a12 * (a12 * a33 - a23 * a13)
          + a13 * (a12 * a23 - a22 * a13))

    lb = jnp.minimum(
        jnp.minimum(a11 - (jnp.abs(a12) + jnp.abs(a13)),
                    a22 - (jnp.abs(a12) + jnp.abs(a23))),
        a33 - (jnp.abs(a13) + jnp.abs(a23)))
    lam = lb
    for _ in range(16):
        pval = ((-lam + c2) * lam - c1) * lam + c0
        pder = (-3.0 * lam + 2.0 * c2) * lam - c1
        denom = jnp.where(jnp.abs(pder) < 1e-30,
                          jnp.where(pder < 0, -1e-30, 1e-30), pder)
        lam = lam - pval / denom

    m11 = a11 - lam
    m22 = a22 - lam
    m33 = a33 - lam
    v1x = a12 * a23 - a13 * m22
    v1y = a13 * a12 - m11 * a23
    v1z = m11 * m22 - a12 * a12
    v2x = a12 * m33 - a13 * a23
    v2y = a13 * a13 - m11 * m33
    v2z = m11 * a23 - a12 * a13
    v3x = m22 * m33 - a23 * a23
    v3y = a23 * a13 - a12 * m33
    v3z = a12 * a23 - m22 * a13
    n1 = v1x * v1x + v1y * v1y + v1z * v1z
    n2 = v2x * v2x + v2y * v2y + v2z * v2z
    n3 = v3x * v3x + v3y * v3y + v3z * v3z
    use2 = n2 > n1
    bx = jnp.where(use2, v2x, v1x)
    by = jnp.where(use2, v2y, v1y)
    bz = jnp.where(use2, v2z, v1z)
    bn = jnp.where(use2, n2, n1)
    use3 = n3 > bn
    bx = jnp.where(use3, v3x, bx)
    by = jnp.where(use3, v3y, by)
    bz = jnp.where(use3, v3z, bz)
    bn = jnp.where(use3, n3, bn)
    inv = lax.rsqrt(jnp.maximum(bn, jnp.float32(1e-38)))
    nx = bx * inv
    ny = by * inv
    nz = bz * inv

    dotv = nx * (-px) + ny * (-py) + nz * (-pz)
    flip = jnp.where(dotv < 0, jnp.float32(-1.0), jnp.float32(1.0))
    nx = nx * flip
    ny = ny * flip
    nz = nz * flip

    desc = jnp.concatenate(
        [px, py, pz, nx, ny, nz, mlx, mly, mlz], axis=1)

    bf = jnp.bfloat16
    h = jnp.dot(desc.astype(bf), w1_ref[...].astype(bf),
                preferred_element_type=jnp.float32)
    h = jnp.maximum(h + b1_ref[...], 0.0)
    h = jnp.dot(h.astype(bf), w2_ref[...].astype(bf),
                preferred_element_type=jnp.float32)
    h = jnp.maximum(h + b2_ref[...], 0.0)
    h = jnp.dot(h.astype(bf), w3_ref[...].astype(bf),
                preferred_element_type=jnp.float32)
    h = h + b3_ref[...]
    out_ref[0] = h.T


@functools.partial(jax.jit, static_argnames=("interpret",))
def _run(point_cloud, vis_mask, W1, b1, W2, b2, W3, b3, interpret=False):
    B, N, _ = point_cloud.shape
    R = 256
    visible = jnp.where(vis_mask[:, :, None], point_cloud,
                        jnp.zeros_like(point_cloud))
    ptsc = jnp.transpose(visible, (0, 2, 1))

    grid = (B, N // R)
    out = pl.pallas_call(
        _geom_body,
        grid=grid,
        in_specs=[
            pl.BlockSpec((1, R, 3), lambda b, r: (b, r, 0)),
            pl.BlockSpec((1, 3, N), lambda b, r: (b, 0, 0)),
            pl.BlockSpec((9, 64), lambda b, r: (0, 0)),
            pl.BlockSpec((1, 64), lambda b, r: (0, 0)),
            pl.BlockSpec((64, 128), lambda b, r: (0, 0)),
            pl.BlockSpec((1, 128), lambda b, r: (0, 0)),
            pl.BlockSpec((128, 256), lambda b, r: (0, 0)),
            pl.BlockSpec((1, 256), lambda b, r: (0, 0)),
        ],
        out_specs=pl.BlockSpec((1, 256, R), lambda b, r: (b, 0, r)),
        out_shape=jax.ShapeDtypeStruct((B, 256, N), jnp.float32),
        interpret=interpret,
    )(visible, ptsc, W1.T, b1[None, :], W2.T, b2[None, :], W3.T,
      b3[None, :])
    return out


def kernel(point_cloud, vis_mask, W1, b1, W2, b2, W3, b3):
    return _run(point_cloud, vis_mask, W1, b1, W2, b2, W3, b3)


# speedup vs baseline: 71.9040x; 1.1752x over previous
import functools

import jax
import jax.numpy as jnp
from jax import lax
from jax.experimental import pallas as pl

K_NN = 20


def _geom_body(ptsr_ref, ptsc_ref, w1_ref, b1_ref, w2_ref, b2_ref,
               w3_ref, b3_ref, out_ref):
    R = ptsr_ref.shape[1]
    N = ptsc_ref.shape[2]
    xr = ptsr_ref[0]
    xc = ptsc_ref[0]

    px = xr[:, 0:1]
    py = xr[:, 1:2]
    pz = xr[:, 2:3]

    dx = xc[0:1, :] - px
    dy = xc[1:2, :] - py
    dz = xc[2:3, :] - pz
    dist = (dx * dx + dy * dy) + dz * dz

    iota = lax.broadcasted_iota(jnp.int32, (R, N), 1)
    bits = lax.bitcast_convert_type(dist, jnp.int32)
    kp1 = jnp.int32(K_NN + 1)
    tpre = jnp.zeros((R, 1), dtype=jnp.int32)
    for b in range(30, -1, -1):
        cand = tpre | jnp.int32(1 << b)
        cnt = jnp.sum((bits < cand).astype(jnp.int32), axis=1, keepdims=True)
        tpre = jnp.where(cnt < kp1, cand, tpre)
    lt = bits < tpre
    cnt_lt = jnp.sum(lt.astype(jnp.int32), axis=1, keepdims=True)
    eq = bits == tpre
    need = kp1 - cnt_lt
    csum = eq.astype(jnp.int32)
    s = 1
    while s < N:
        shifted = jnp.concatenate(
            [jnp.zeros((R, s), jnp.int32), csum[:, :N - s]], axis=1)
        csum = csum + shifted
        s *= 2
    sel = jnp.logical_or(lt, jnp.logical_and(eq, csum <= need))
    tmin = jnp.min(bits, axis=1, keepdims=True)
    idx0 = jnp.min(jnp.where(bits == tmin, iota, N), axis=1, keepdims=True)
    sel = jnp.logical_and(sel, iota != idx0)

    zero = jnp.float32(0.0)

    def msum(v):
        return jnp.sum(jnp.where(sel, v, zero), axis=1, keepdims=True)

    kf = jnp.float32(K_NN)
    mlx = msum(dx) / kf
    mly = msum(dy) / kf
    mlz = msum(dz) / kf
    cx = (dx - mlx).astype(jnp.bfloat16).astype(jnp.float32)
    cy = (dy - mly).astype(jnp.bfloat16).astype(jnp.float32)
    cz = (dz - mlz).astype(jnp.bfloat16).astype(jnp.float32)
    a11 = msum(cx * cx)
    a22 = msum(cy * cy)
    a33 = msum(cz * cz)
    a12 = msum(cx * cy)
    a13 = msum(cx * cz)
    a23 = msum(cy * cz)

    scale = jnp.maximum(
        jnp.maximum(jnp.maximum(jnp.abs(a11), jnp.abs(a22)),
                    jnp.maximum(jnp.abs(a33), jnp.abs(a12))),
        jnp.maximum(jnp.abs(a13), jnp.abs(a23)))
    scale = jnp.maximum(scale, jnp.float32(1e-30))
    a11 = a11 / scale
    a22 = a22 / scale
    a33 = a33 / scale
    a12 = a12 / scale
    a13 = a13 / scale
    a23 = a23 / scale

    c2 = a11 + a22 + a33
    c1 = (a11 * a22 + a11 * a33 + a22 * a33) - (a12 * a12 + a13 * a13 + a23 * a23)
    c0 = (a11 * (a22 * a33 - a23 * a23)
          - a12 * (a12 * a33 - a23 * a13)
          + a13 * (a12 * a23 - a22 * a13))

    lb = jnp.minimum(
        jnp.minimum(a11 - (jnp.abs(a12) + jnp.abs(a13)),
                    a22 - (jnp.abs(a12) + jnp.abs(a23))),
        a33 - (jnp.abs(a13) + jnp.abs(a23)))
    lam = lb
    for _ in range(16):
        pval = ((-lam + c2) * lam - c1) * lam + c0
        pder = (-3.0 * lam + 2.0 * c2) * lam - c1
        denom = jnp.where(jnp.abs(pder) < 1e-30,
                          jnp.where(pder < 0, -1e-30, 1e-30), pder)
        lam = lam - pval / denom

    m11 = a11 - lam
    m22 = a22 - lam
    m33 = a33 - lam
    v1x = a12 * a23 - a13 * m22
    v1y = a13 * a12 - m11 * a23
    v1z = m11 * m22 - a12 * a12
    v2x = a12 * m33 - a13 * a23
    v2y = a13 * a13 - m11 * m33
    v2z = m11 * a23 - a12 * a13
    v3x = m22 * m33 - a23 * a23
    v3y = a23 * a13 - a12 * m33
    v3z = a12 * a23 - m22 * a13
    n1 = v1x * v1x + v1y * v1y + v1z * v1z
    n2 = v2x * v2x + v2y * v2y + v2z * v2z
    n3 = v3x * v3x + v3y * v3y + v3z * v3z
    use2 = n2 > n1
    bx = jnp.where(use2, v2x, v1x)
    by = jnp.where(use2, v2y, v1y)
    bz = jnp.where(use2, v2z, v1z)
    bn = jnp.where(use2, n2, n1)
    use3 = n3 > bn
    bx = jnp.where(use3, v3x, bx)
    by = jnp.where(use3, v3y, by)
    bz = jnp.where(use3, v3z, bz)
    bn = jnp.where(use3, n3, bn)
    inv = lax.rsqrt(jnp.maximum(bn, jnp.float32(1e-38)))
    nx = bx * inv
    ny = by * inv
    nz = bz * inv

    dotv = nx * (-px) + ny * (-py) + nz * (-pz)
    flip = jnp.where(dotv < 0, jnp.float32(-1.0), jnp.float32(1.0))
    nx = nx * flip
    ny = ny * flip
    nz = nz * flip

    desc = jnp.concatenate(
        [px, py, pz, nx, ny, nz, mlx, mly, mlz], axis=1)

    bf = jnp.bfloat16
    h = jnp.dot(desc.astype(bf), w1_ref[...].astype(bf),
                preferred_element_type=jnp.float32)
    h = jnp.maximum(h + b1_ref[...], 0.0)
    h = jnp.dot(h.astype(bf), w2_ref[...].astype(bf),
                preferred_element_type=jnp.float32)
    h = jnp.maximum(h + b2_ref[...], 0.0)
    h = jnp.dot(h.astype(bf), w3_ref[...].astype(bf),
                preferred_element_type=jnp.float32)
    h = h + b3_ref[...]
    out_ref[0] = h.T


@functools.partial(jax.jit, static_argnames=("interpret",))
def _run(point_cloud, vis_mask, W1, b1, W2, b2, W3, b3, interpret=False):
    B, N, _ = point_cloud.shape
    R = 256
    visible = jnp.where(vis_mask[:, :, None], point_cloud,
                        jnp.zeros_like(point_cloud))
    ptsc = jnp.transpose(visible, (0, 2, 1))

    grid = (B, N // R)
    out = pl.pallas_call(
        _geom_body,
        grid=grid,
        in_specs=[
            pl.BlockSpec((1, R, 3), lambda b, r: (b, r, 0)),
            pl.BlockSpec((1, 3, N), lambda b, r: (b, 0, 0)),
            pl.BlockSpec((9, 64), lambda b, r: (0, 0)),
            pl.BlockSpec((1, 64), lambda b, r: (0, 0)),
            pl.BlockSpec((64, 128), lambda b, r: (0, 0)),
            pl.BlockSpec((1, 128), lambda b, r: (0, 0)),
            pl.BlockSpec((128, 256), lambda b, r: (0, 0)),
            pl.BlockSpec((1, 256), lambda b, r: (0, 0)),
        ],
        out_specs=pl.BlockSpec((1, 256, R), lambda b, r: (b, 0, r)),
        out_shape=jax.ShapeDtypeStruct((B, 256, N), jnp.float32),
        interpret=interpret,
    )(visible, ptsc, W1.T, b1[None, :], W2.T, b2[None, :], W3.T,
      b3[None, :])
    return out


def kernel(point_cloud, vis_mask, W1, b1, W2, b2, W3, b3):
    return _run(point_cloud, vis_mask, W1, b1, W2, b2, W3, b3)
